# baseline (speedup 1.0000x reference)
import functools

import jax
import jax.numpy as jnp
from jax import lax
from jax.experimental import pallas as pl
from jax.experimental.pallas import tpu as pltpu
from jax.experimental.pallas import tpu_sc as plsc

DECIM = 3
LANES = 16

B = 16
N_IN = 480000
N_OUT = 160000

NUM_CORES = 2
NUM_SUBCORES = 16
NUM_WORKERS = NUM_CORES * NUM_SUBCORES

HALVES = NUM_WORKERS // B
OUT_PER_WORKER = N_OUT // HALVES
IN_PER_WORKER = OUT_PER_WORKER * DECIM

NO_CHUNK = 9600
NI_CHUNK = NO_CHUNK * DECIM
OUT_CHUNKS = (3200,) + (9600,) * 8
NBUF = 3
UNROLL = 16


def _body(wav_hbm, out_hbm, in_buf0, in_buf1, in_buf2, out_buf0, out_buf1,
          out_buf2, sem_in0, sem_in1, sem_in2, sem_out0, sem_out1, sem_out2):
    wid = lax.axis_index("s") * NUM_CORES + lax.axis_index("c")
    row = wid // HALVES
    half = wid % HALVES
    idx0 = lax.iota(jnp.int32, LANES) * DECIM
    in_bufs = (in_buf0, in_buf1, in_buf2)
    out_bufs = (out_buf0, out_buf1, out_buf2)
    sems_in = (sem_in0, sem_in1, sem_in2)
    sems_out = (sem_out0, sem_out1, sem_out2)
    out_offs = [0]
    for n in OUT_CHUNKS:
        out_offs.append(out_offs[-1] + n)
    num_chunks = len(OUT_CHUNKS)

    def in_copy(k, slot):
        src = wav_hbm.at[row, pl.ds(half * IN_PER_WORKER +
                                    out_offs[k] * DECIM,
                                    OUT_CHUNKS[k] * DECIM)]
        dst = in_bufs[slot].at[pl.ds(0, OUT_CHUNKS[k] * DECIM)]
        return pltpu.make_async_copy(src, dst, sems_in[slot])

    def out_copy(k, slot):
        dst = out_hbm.at[row, pl.ds(half * OUT_PER_WORKER + out_offs[k],
                                    OUT_CHUNKS[k])]
        src = out_bufs[slot].at[pl.ds(0, OUT_CHUNKS[k])]
        return pltpu.make_async_copy(src, dst, sems_out[slot])

    pending_out = [None] * NBUF
    for k in range(min(NBUF - 1, num_chunks)):
        in_copy(k, k % NBUF).start()
    for k in range(num_chunks):
        slot = k % NBUF
        if k + NBUF - 1 < num_chunks:
            in_copy(k + NBUF - 1, (k + NBUF - 1) % NBUF).start()
        in_copy(k, slot).wait()
        if pending_out[slot] is not None:
            pending_out[slot].wait()
        in_ref = in_bufs[slot]
        out_ref = out_bufs[slot]

        @plsc.parallel_loop(0, OUT_CHUNKS[k] // LANES, unroll=UNROLL)
        def _(j):
            idx = idx0 + j * (LANES * DECIM)
            out_ref[pl.ds(j * LANES, LANES)] = plsc.load_gather(in_ref, [idx])

        oc = out_copy(k, slot)
        oc.start()
        pending_out[slot] = oc
    for oc in pending_out:
        if oc is not None:
            oc.wait()


@jax.jit
def kernel(wav):
    wav = wav.reshape(wav.shape[0], -1)
    assert wav.shape == (B, N_IN), wav.shape
    mesh = plsc.VectorSubcoreMesh(core_axis_name="c", subcore_axis_name="s")
    run = functools.partial(
        pl.kernel,
        mesh=mesh,
        out_type=jax.ShapeDtypeStruct((B, N_OUT), jnp.float32),
        scratch_types=[
            pltpu.VMEM((NI_CHUNK,), jnp.float32),
            pltpu.VMEM((NI_CHUNK,), jnp.float32),
            pltpu.VMEM((NI_CHUNK,), jnp.float32),
            pltpu.VMEM((NO_CHUNK,), jnp.float32),
            pltpu.VMEM((NO_CHUNK,), jnp.float32),
            pltpu.VMEM((NO_CHUNK,), jnp.float32),
            pltpu.SemaphoreType.DMA,
            pltpu.SemaphoreType.DMA,
            pltpu.SemaphoreType.DMA,
            pltpu.SemaphoreType.DMA,
            pltpu.SemaphoreType.DMA,
            pltpu.SemaphoreType.DMA,
        ],
        compiler_params=pltpu.CompilerParams(
            needs_layout_passes=False,
            disable_bounds_checks=True,
            disable_semaphore_checks=True,
            skip_device_barrier=True,
        ),
    )(_body)
    return run(wav)


# speedup vs baseline: 1.0150x; 1.0150x over previous
import functools

import jax
import jax.numpy as jnp
from jax import lax
from jax.experimental import pallas as pl
from jax.experimental.pallas import tpu as pltpu
from jax.experimental.pallas import tpu_sc as plsc

DECIM = 3
LANES = 16

B = 16
N_IN = 480000
N_OUT = 160000

NUM_CORES = 2
NUM_SUBCORES = 16
NUM_WORKERS = NUM_CORES * NUM_SUBCORES

HALVES = NUM_WORKERS // B
OUT_PER_WORKER = N_OUT // HALVES
IN_PER_WORKER = OUT_PER_WORKER * DECIM

NO_CHUNK = 16000
NI_CHUNK = NO_CHUNK * DECIM
OUT_CHUNKS = (3200, 12800, 16000, 16000, 16000, 16000)
NBUF = 2
UNROLL = 4


def _body(wav_hbm, out_hbm, in_buf0, in_buf1, out_buf0, out_buf1, sem_in0,
          sem_in1, sem_out0, sem_out1):
    wid = lax.axis_index("s") * NUM_CORES + lax.axis_index("c")
    row = wid // HALVES
    half = wid % HALVES
    idx0 = lax.iota(jnp.int32, LANES) * DECIM
    in_bufs = (in_buf0, in_buf1)
    out_bufs = (out_buf0, out_buf1)
    sems_in = (sem_in0, sem_in1)
    sems_out = (sem_out0, sem_out1)
    out_offs = [0]
    for n in OUT_CHUNKS:
        out_offs.append(out_offs[-1] + n)
    num_chunks = len(OUT_CHUNKS)

    def in_copy(k, slot):
        src = wav_hbm.at[row, pl.ds(half * IN_PER_WORKER +
                                    out_offs[k] * DECIM,
                                    OUT_CHUNKS[k] * DECIM)]
        dst = in_bufs[slot].at[pl.ds(0, OUT_CHUNKS[k] * DECIM)]
        return pltpu.make_async_copy(src, dst, sems_in[slot])

    def out_copy(k, slot):
        dst = out_hbm.at[row, pl.ds(half * OUT_PER_WORKER + out_offs[k],
                                    OUT_CHUNKS[k])]
        src = out_bufs[slot].at[pl.ds(0, OUT_CHUNKS[k])]
        return pltpu.make_async_copy(src, dst, sems_out[slot])

    pending_out = [None] * NBUF
    for k in range(min(NBUF - 1, num_chunks)):
        in_copy(k, k % NBUF).start()
    for k in range(num_chunks):
        slot = k % NBUF
        if k + NBUF - 1 < num_chunks:
            in_copy(k + NBUF - 1, (k + NBUF - 1) % NBUF).start()
        in_copy(k, slot).wait()
        if pending_out[slot] is not None:
            pending_out[slot].wait()
        in_ref = in_bufs[slot]
        out_ref = out_bufs[slot]

        @plsc.parallel_loop(0, OUT_CHUNKS[k] // LANES, unroll=UNROLL)
        def _(j):
            idx = idx0 + j * (LANES * DECIM)
            out_ref[pl.ds(j * LANES, LANES)] = plsc.load_gather(in_ref, [idx])

        oc = out_copy(k, slot)
        oc.start()
        pending_out[slot] = oc
    for oc in pending_out:
        if oc is not None:
            oc.wait()


@jax.jit
def kernel(wav):
    wav = wav.reshape(wav.shape[0], -1)
    assert wav.shape == (B, N_IN), wav.shape
    mesh = plsc.VectorSubcoreMesh(core_axis_name="c", subcore_axis_name="s")
    run = functools.partial(
        pl.kernel,
        mesh=mesh,
        out_type=jax.ShapeDtypeStruct((B, N_OUT), jnp.float32),
        scratch_types=[
            pltpu.VMEM((NI_CHUNK,), jnp.float32),
            pltpu.VMEM((NI_CHUNK,), jnp.float32),
            pltpu.VMEM((NO_CHUNK,), jnp.float32),
            pltpu.VMEM((NO_CHUNK,), jnp.float32),
            pltpu.SemaphoreType.DMA,
            pltpu.SemaphoreType.DMA,
            pltpu.SemaphoreType.DMA,
            pltpu.SemaphoreType.DMA,
        ],
        compiler_params=pltpu.CompilerParams(
            needs_layout_passes=False,
            disable_bounds_checks=True,
            disable_semaphore_checks=True,
            skip_device_barrier=True,
        ),
    )(_body)
    return run(wav)


# speedup vs baseline: 1.0188x; 1.0037x over previous
import functools

import jax
import jax.numpy as jnp
from jax import lax
from jax.experimental import pallas as pl
from jax.experimental.pallas import tpu as pltpu
from jax.experimental.pallas import tpu_sc as plsc

DECIM = 3
LANES = 16

B = 16
N_IN = 480000
N_OUT = 160000

NUM_CORES = 2
NUM_SUBCORES = 16
NUM_WORKERS = NUM_CORES * NUM_SUBCORES

HALVES = NUM_WORKERS // B
OUT_PER_WORKER = N_OUT // HALVES
IN_PER_WORKER = OUT_PER_WORKER * DECIM

NO_CHUNK = 16000
NI_CHUNK = NO_CHUNK * DECIM
OUT_CHUNKS = (3200, 12800, 16000, 16000, 16000, 16000)
NBUF = 2
UNROLL = 8


def _body(wav_hbm, out_hbm, in_buf0, in_buf1, out_buf0, out_buf1, sem_in0,
          sem_in1, sem_out0, sem_out1):
    wid = lax.axis_index("s") * NUM_CORES + lax.axis_index("c")
    row = wid // HALVES
    half = wid % HALVES
    idx0 = lax.iota(jnp.int32, LANES) * DECIM
    in_bufs = (in_buf0, in_buf1)
    out_bufs = (out_buf0, out_buf1)
    sems_in = (sem_in0, sem_in1)
    sems_out = (sem_out0, sem_out1)
    out_offs = [0]
    for n in OUT_CHUNKS:
        out_offs.append(out_offs[-1] + n)
    num_chunks = len(OUT_CHUNKS)

    def in_copy(k, slot):
        src = wav_hbm.at[row, pl.ds(half * IN_PER_WORKER +
                                    out_offs[k] * DECIM,
                                    OUT_CHUNKS[k] * DECIM)]
        dst = in_bufs[slot].at[pl.ds(0, OUT_CHUNKS[k] * DECIM)]
        return pltpu.make_async_copy(src, dst, sems_in[slot])

    def out_copy(k, slot):
        dst = out_hbm.at[row, pl.ds(half * OUT_PER_WORKER + out_offs[k],
                                    OUT_CHUNKS[k])]
        src = out_bufs[slot].at[pl.ds(0, OUT_CHUNKS[k])]
        return pltpu.make_async_copy(src, dst, sems_out[slot])

    pending_out = [None] * NBUF
    for k in range(min(NBUF - 1, num_chunks)):
        in_copy(k, k % NBUF).start()
    for k in range(num_chunks):
        slot = k % NBUF
        if k + NBUF - 1 < num_chunks:
            in_copy(k + NBUF - 1, (k + NBUF - 1) % NBUF).start()
        in_copy(k, slot).wait()
        if pending_out[slot] is not None:
            pending_out[slot].wait()
        in_ref = in_bufs[slot]
        out_ref = out_bufs[slot]

        @plsc.parallel_loop(0, OUT_CHUNKS[k] // LANES, unroll=UNROLL)
        def _(j):
            idx = idx0 + j * (LANES * DECIM)
            out_ref[pl.ds(j * LANES, LANES)] = plsc.load_gather(in_ref, [idx])

        oc = out_copy(k, slot)
        oc.start()
        pending_out[slot] = oc
    for oc in pending_out:
        if oc is not None:
            oc.wait()


@jax.jit
def kernel(wav):
    wav = wav.reshape(wav.shape[0], -1)
    assert wav.shape == (B, N_IN), wav.shape
    mesh = plsc.VectorSubcoreMesh(core_axis_name="c", subcore_axis_name="s")
    run = functools.partial(
        pl.kernel,
        mesh=mesh,
        out_type=jax.ShapeDtypeStruct((B, N_OUT), jnp.float32),
        scratch_types=[
            pltpu.VMEM((NI_CHUNK,), jnp.float32),
            pltpu.VMEM((NI_CHUNK,), jnp.float32),
            pltpu.VMEM((NO_CHUNK,), jnp.float32),
            pltpu.VMEM((NO_CHUNK,), jnp.float32),
            pltpu.SemaphoreType.DMA,
            pltpu.SemaphoreType.DMA,
            pltpu.SemaphoreType.DMA,
            pltpu.SemaphoreType.DMA,
        ],
        compiler_params=pltpu.CompilerParams(
            needs_layout_passes=False,
            disable_bounds_checks=True,
            disable_semaphore_checks=True,
            skip_device_barrier=True,
        ),
    )(_body)
    return run(wav)
